# baseline (speedup 1.0000x reference)
import functools

import jax
import jax.numpy as jnp
from jax import lax
from jax.experimental import pallas as pl
from jax.experimental.pallas import tpu as pltpu
from jax.experimental.pallas import tpu_sc as plsc

NSAMP = 1024
NEXP = 8192
KT = 32
LSEQ = 200

NWORKER = 32
ROWS_PER = NSAMP // NWORKER
CHUNK = 128
NCH = NEXP // CHUNK


def _stage_ab(ev_ref, t_ref, alpha_ref, mu_ref, bu_ref, exp_ref, prm_in_ref,
              st_ref, thr_ref, prm_ref, w_ref):
    tle = prm_in_ref[0, 0]
    bnd = prm_in_ref[0, 1]
    beta = prm_in_ref[0, 2]
    onehot_t = (jax.lax.broadcasted_iota(jnp.int32, (KT, LSEQ), 0)
                == ev_ref[:, :]).astype(jnp.float32)
    w0 = 0.1 * jnp.exp(-beta * (tle - t_ref[:, :]))
    s_col = jax.lax.dot_general(onehot_t, w0, (((1,), (1,)), ((), ())),
                                preferred_element_type=jnp.float32)
    c_col = jax.lax.dot_general(alpha_ref[:, :], s_col, (((0,), (0,)), ((), ())),
                                preferred_element_type=jnp.float32)

    def total_intensity(ebs):
        acc = jnp.zeros_like(ebs)
        for k in range(KT):
            acc = acc + jax.nn.softplus(mu_ref[0, k] + ebs * c_col[k, 0])
        return acc + KT * 1e-6

    tfb_rel = bu_ref[:, :] * (bnd - tle)
    sums = total_intensity(jnp.exp(-beta * tfb_rel))
    rate = jnp.max(sums) * 5.0
    e_row = -jnp.log1p(-exp_ref[:, :]) / rate
    e = jnp.concatenate([e_row[:, i * 128:(i + 1) * 128] for i in range(64)],
                        axis=0)
    ii = jax.lax.broadcasted_iota(jnp.int32, (128, 128), 0)
    jj = jax.lax.broadcasted_iota(jnp.int32, (128, 128), 1)
    upper = (ii <= jj).astype(jnp.float32)
    y = jnp.dot(e, upper, preferred_element_type=jnp.float32)
    totals = y[:, 127:128]
    i2 = jax.lax.broadcasted_iota(jnp.int32, (64, 64), 0)
    j2 = jax.lax.broadcasted_iota(jnp.int32, (64, 64), 1)
    lstrict = (j2 < i2).astype(jnp.float32)
    off = jnp.dot(lstrict, totals, preferred_element_type=jnp.float32)
    strel = y + off
    st_ref[:, :] = strel + tle
    ti = total_intensity(jnp.exp(-beta * strel))
    thr_ref[:, :] = ti / rate
    stlast = strel[63:64, 127:128] + tle
    bigv = stlast + 1.0
    fbv = jnp.where(stlast > bnd, stlast, bnd)
    lane = jax.lax.broadcasted_iota(jnp.int32, (1, 16), 1)
    prm_ref[:, :] = jnp.where(lane == 0, bigv, jnp.where(lane == 1, fbv, 0.0))
    w_ref[:, :] = jnp.full((8, 128), 1.0 / NSAMP, jnp.float32)


def _vmin16(v):
    vals = [v[j] for j in range(16)]
    while len(vals) > 1:
        vals = [jnp.minimum(vals[i], vals[i + 1]) for i in range(0, len(vals), 2)]
    return vals[0]


def _vmax16(v):
    vals = [v[j] for j in range(16)]
    while len(vals) > 1:
        vals = [jnp.maximum(vals[i], vals[i + 1]) for i in range(0, len(vals), 2)]
    return vals[0]


def _make_sc_scan():
    mesh = plsc.VectorSubcoreMesh(core_axis_name="c", subcore_axis_name="s")

    @functools.partial(
        pl.kernel, mesh=mesh,
        out_type=jax.ShapeDtypeStruct((NSAMP,), jnp.float32),
        scratch_types=[
            pltpu.VMEM((1, 128), jnp.float32),
            pltpu.VMEM((1, 128), jnp.float32),
            pltpu.VMEM((ROWS_PER, CHUNK), jnp.float32),
            pltpu.VMEM((1, 16), jnp.float32),
            pltpu.VMEM((ROWS_PER,), jnp.float32),
            pltpu.SemaphoreType.DMA,
        ],
    )
    def scan_k(u_hbm, thr_hbm, st_hbm, prm_hbm, out_hbm,
               thr_v, st_v, u_v, prm_v, res_v, sem):
        wid = lax.axis_index("s") * 2 + lax.axis_index("c")
        base = wid * ROWS_PER

        def start_dmas(c):
            cp_u = pltpu.make_async_copy(
                u_hbm.at[pl.ds(base, ROWS_PER), pl.ds(c * CHUNK, CHUNK)],
                u_v, sem)
            cp_t = pltpu.make_async_copy(
                thr_hbm.at[pl.ds(c, 1), :], thr_v, sem)
            cp_s = pltpu.make_async_copy(
                st_hbm.at[pl.ds(c, 1), :], st_v, sem)
            cp_u.start()
            cp_t.start()
            cp_s.start()
            return cp_u, cp_t, cp_s

        def wait_dmas(cps):
            for cp in cps:
                cp.wait()

        cps0 = start_dmas(0)
        pltpu.sync_copy(prm_hbm, prm_v)
        pv = prm_v[0, pl.ds(0, 16)]
        big = pv[0]
        fb = pv[1]
        bigv = jnp.full((16,), big, jnp.float32)
        iota16 = lax.iota(jnp.int32, 16)

        def scan_chunk(r0, r1):
            tvs = [thr_v[0, pl.ds(k * 16, 16)] for k in range(CHUNK // 16)]
            svs = [st_v[0, pl.ds(k * 16, 16)] for k in range(CHUNK // 16)]

            def row_body(r, rc):
                a0, a1 = rc
                accv = bigv
                for k in range(CHUNK // 16):
                    uv = u_v[r, pl.ds(k * 16, 16)]
                    accv = jnp.minimum(
                        accv, jnp.where(uv < tvs[k], svs[k], bigv))
                rowmin = _vmin16(accv)
                u0 = jnp.where(iota16 == r, jnp.minimum(a0, rowmin), a0)
                u1 = jnp.where(iota16 == (r - 16), jnp.minimum(a1, rowmin), a1)
                return (u0, u1)

            r0, r1 = lax.fori_loop(0, ROWS_PER, row_body, (r0, r1))
            res_v[pl.ds(0, 16)] = r0
            res_v[pl.ds(16, 16)] = r1
            return jnp.int32(_vmax16(jnp.maximum(r0, r1)) < big)

        wait_dmas(cps0)
        done0 = scan_chunk(bigv, bigv)

        def chunk_work(c):
            wait_dmas(start_dmas(c))
            return scan_chunk(res_v[pl.ds(0, 16)], res_v[pl.ds(16, 16)])

        def chunk_body(c, done):
            return lax.cond(done == 0,
                            lambda: chunk_work(c),
                            lambda: done)

        lax.fori_loop(1, NCH, chunk_body, done0)
        r0 = res_v[pl.ds(0, 16)]
        r1 = res_v[pl.ds(16, 16)]
        res_v[pl.ds(0, 16)] = jnp.where(r0 < bigv, r0, fb)
        res_v[pl.ds(16, 16)] = jnp.where(r1 < bigv, r1, fb)
        pltpu.sync_copy(res_v, out_hbm.at[pl.ds(base, ROWS_PER)])

    return scan_k


def kernel(event_seq, time_seq, time_last_event, boundary, bound_u, exp_u,
           unif_numbers, mu, alpha, beta_raw):
    f32 = jnp.float32
    tle = jnp.float32(time_last_event)
    bnd = jnp.float32(boundary)
    beta = jnp.abs(beta_raw[0]) + 0.1
    prm_a = jnp.stack([tle, bnd, beta]).reshape(1, 3).astype(f32)

    st2d, thr2d, prm16, w2d = pl.pallas_call(
        _stage_ab,
        out_shape=[
            jax.ShapeDtypeStruct((64, 128), f32),
            jax.ShapeDtypeStruct((64, 128), f32),
            jax.ShapeDtypeStruct((1, 16), f32),
            jax.ShapeDtypeStruct((8, 128), f32),
        ],
    )(event_seq.astype(jnp.int32), time_seq, alpha, mu.reshape(1, KT),
      bound_u.reshape(1, 10), exp_u, prm_a)

    rst = _make_sc_scan()(unif_numbers, thr2d, st2d, prm16)
    return rst, w2d.reshape(NSAMP)


# speedup vs baseline: 1.0042x; 1.0042x over previous
import functools

import jax
import jax.numpy as jnp
from jax import lax
from jax.experimental import pallas as pl
from jax.experimental.pallas import tpu as pltpu
from jax.experimental.pallas import tpu_sc as plsc

NSAMP = 1024
NEXP = 8192
KT = 32
LSEQ = 200

NWORKER = 32
ROWS_PER = NSAMP // NWORKER
CHUNK = 128
NCH = NEXP // CHUNK


def _stage_ab(ev_ref, t_ref, alpha_ref, mu_ref, bu_ref, exp_ref, prm_in_ref,
              st_ref, thr_ref, prm_ref, w_ref):
    tle = prm_in_ref[0, 0]
    bnd = prm_in_ref[0, 1]
    beta = prm_in_ref[0, 2]
    onehot_t = (jax.lax.broadcasted_iota(jnp.int32, (KT, LSEQ), 0)
                == ev_ref[:, :]).astype(jnp.float32)
    w0 = 0.1 * jnp.exp(-beta * (tle - t_ref[:, :]))
    s_col = jax.lax.dot_general(onehot_t, w0, (((1,), (1,)), ((), ())),
                                preferred_element_type=jnp.float32)
    c_col = jax.lax.dot_general(alpha_ref[:, :], s_col, (((0,), (0,)), ((), ())),
                                preferred_element_type=jnp.float32)

    def total_intensity(ebs):
        acc = jnp.zeros_like(ebs)
        for k in range(KT):
            acc = acc + jax.nn.softplus(mu_ref[0, k] + ebs * c_col[k, 0])
        return acc + KT * 1e-6

    tfb_rel = bu_ref[:, :] * (bnd - tle)
    sums = total_intensity(jnp.exp(-beta * tfb_rel))
    rate = jnp.max(sums) * 5.0
    e_row = -jnp.log1p(-exp_ref[:, :]) / rate
    e = jnp.concatenate([e_row[:, i * 128:(i + 1) * 128] for i in range(64)],
                        axis=0)
    ii = jax.lax.broadcasted_iota(jnp.int32, (128, 128), 0)
    jj = jax.lax.broadcasted_iota(jnp.int32, (128, 128), 1)
    upper = (ii <= jj).astype(jnp.float32)
    y = jnp.dot(e, upper, preferred_element_type=jnp.float32)
    totals = y[:, 127:128]
    i2 = jax.lax.broadcasted_iota(jnp.int32, (64, 64), 0)
    j2 = jax.lax.broadcasted_iota(jnp.int32, (64, 64), 1)
    lstrict = (j2 < i2).astype(jnp.float32)
    off = jnp.dot(lstrict, totals, preferred_element_type=jnp.float32)
    strel = y + off
    st_ref[:, :] = strel + tle
    ti = total_intensity(jnp.exp(-beta * strel))
    thr_ref[:, :] = ti / rate
    stlast = strel[63:64, 127:128] + tle
    bigv = stlast + 1.0
    fbv = jnp.where(stlast > bnd, stlast, bnd)
    lane = jax.lax.broadcasted_iota(jnp.int32, (1, 16), 1)
    prm_ref[:, :] = jnp.where(lane == 0, bigv, jnp.where(lane == 1, fbv, 0.0))
    w_ref[:, :] = jnp.full((8, 128), 1.0 / NSAMP, jnp.float32)


def _vmin16(v):
    vals = [v[j] for j in range(16)]
    while len(vals) > 1:
        vals = [jnp.minimum(vals[i], vals[i + 1]) for i in range(0, len(vals), 2)]
    return vals[0]


def _vmax16(v):
    vals = [v[j] for j in range(16)]
    while len(vals) > 1:
        vals = [jnp.maximum(vals[i], vals[i + 1]) for i in range(0, len(vals), 2)]
    return vals[0]


def _make_sc_scan():
    mesh = plsc.VectorSubcoreMesh(core_axis_name="c", subcore_axis_name="s")

    @functools.partial(
        pl.kernel, mesh=mesh,
        out_type=jax.ShapeDtypeStruct((NSAMP,), jnp.float32),
        scratch_types=[
            pltpu.VMEM((1, 128), jnp.float32),
            pltpu.VMEM((1, 128), jnp.float32),
            pltpu.VMEM((ROWS_PER, CHUNK), jnp.float32),
            pltpu.VMEM((1, 16), jnp.float32),
            pltpu.VMEM((ROWS_PER,), jnp.float32),
            pltpu.SemaphoreType.DMA,
        ],
    )
    def scan_k(u_hbm, thr_hbm, st_hbm, prm_hbm, out_hbm,
               thr_v, st_v, u_v, prm_v, res_v, sem):
        wid = lax.axis_index("s") * 2 + lax.axis_index("c")
        base = wid * ROWS_PER

        def start_dmas(c):
            cp_u = pltpu.make_async_copy(
                u_hbm.at[pl.ds(base, ROWS_PER), pl.ds(c * CHUNK, CHUNK)],
                u_v, sem)
            cp_t = pltpu.make_async_copy(
                thr_hbm.at[pl.ds(c, 1), :], thr_v, sem)
            cp_s = pltpu.make_async_copy(
                st_hbm.at[pl.ds(c, 1), :], st_v, sem)
            cp_u.start()
            cp_t.start()
            cp_s.start()
            return cp_u, cp_t, cp_s

        def wait_dmas(cps):
            for cp in cps:
                cp.wait()

        cps0 = start_dmas(0)
        pltpu.sync_copy(prm_hbm, prm_v)
        pv = prm_v[0, pl.ds(0, 16)]
        big = pv[0]
        fb = pv[1]
        bigv = jnp.full((16,), big, jnp.float32)
        iota16 = lax.iota(jnp.int32, 16)

        def scan_chunk(r0, r1):
            def row_body(r, rc):
                a0, a1 = rc
                accv = bigv
                for k in range(CHUNK // 16):
                    uv = u_v[r, pl.ds(k * 16, 16)]
                    tv = thr_v[0, pl.ds(k * 16, 16)]
                    sv = st_v[0, pl.ds(k * 16, 16)]
                    accv = jnp.minimum(accv, jnp.where(uv < tv, sv, bigv))
                rowmin = _vmin16(accv)
                u0 = jnp.where(iota16 == r, jnp.minimum(a0, rowmin), a0)
                u1 = jnp.where(iota16 == (r - 16), jnp.minimum(a1, rowmin), a1)
                return (u0, u1)

            r0, r1 = lax.fori_loop(0, ROWS_PER, row_body, (r0, r1))
            res_v[pl.ds(0, 16)] = r0
            res_v[pl.ds(16, 16)] = r1
            return jnp.int32(_vmax16(jnp.maximum(r0, r1)) < big)

        wait_dmas(cps0)
        done0 = scan_chunk(bigv, bigv)

        def chunk_work(c):
            wait_dmas(start_dmas(c))
            return scan_chunk(res_v[pl.ds(0, 16)], res_v[pl.ds(16, 16)])

        def chunk_body(c, done):
            return lax.cond(done == 0,
                            lambda: chunk_work(c),
                            lambda: done)

        lax.fori_loop(1, NCH, chunk_body, done0)
        r0 = res_v[pl.ds(0, 16)]
        r1 = res_v[pl.ds(16, 16)]
        res_v[pl.ds(0, 16)] = jnp.where(r0 < bigv, r0, fb)
        res_v[pl.ds(16, 16)] = jnp.where(r1 < bigv, r1, fb)
        pltpu.sync_copy(res_v, out_hbm.at[pl.ds(base, ROWS_PER)])

    return scan_k


def kernel(event_seq, time_seq, time_last_event, boundary, bound_u, exp_u,
           unif_numbers, mu, alpha, beta_raw):
    f32 = jnp.float32
    tle = jnp.float32(time_last_event)
    bnd = jnp.float32(boundary)
    beta = jnp.abs(beta_raw[0]) + 0.1
    prm_a = jnp.stack([tle, bnd, beta]).reshape(1, 3).astype(f32)

    st2d, thr2d, prm16, w2d = pl.pallas_call(
        _stage_ab,
        out_shape=[
            jax.ShapeDtypeStruct((64, 128), f32),
            jax.ShapeDtypeStruct((64, 128), f32),
            jax.ShapeDtypeStruct((1, 16), f32),
            jax.ShapeDtypeStruct((8, 128), f32),
        ],
    )(event_seq.astype(jnp.int32), time_seq, alpha, mu.reshape(1, KT),
      bound_u.reshape(1, 10), exp_u, prm_a)

    rst = _make_sc_scan()(unif_numbers, thr2d, st2d, prm16)
    return rst, w2d.reshape(NSAMP)
